# baseline (speedup 1.0000x reference)
import functools

import jax
import jax.numpy as jnp
from jax import lax
from jax.experimental import pallas as pl
from jax.experimental.pallas import tpu as pltpu, tpu_sc as plsc

N = 1024
D = 768

_NC = 2
_NS = 16
_NW = _NC * _NS
_RPW = N // _NW


@functools.cache
def _build_pe_gather():
    @functools.partial(
        pl.kernel,
        mesh=plsc.VectorSubcoreMesh(
            core_axis_name="c", subcore_axis_name="s",
            num_cores=_NC, num_subcores=_NS),
        out_type=jax.ShapeDtypeStruct((N, D), jnp.float32),
        scratch_types=[
            pltpu.VMEM((_RPW,), jnp.int32),
            pltpu.VMEM((_RPW, D), jnp.float32),
            pltpu.SemaphoreType.DMA,
        ],
    )
    def _pe_gather(pe_hbm, out_hbm, idx_v, rows_v, sem):
        wid = lax.axis_index("s") * _NC + lax.axis_index("c")
        base = wid * _RPW
        for j in range(_RPW // 16):
            idx_v[pl.ds(16 * j, 16)] = (
                lax.iota(jnp.int32, 16) + (base + 16 * j))
        pltpu.async_copy(pe_hbm.at[idx_v], rows_v, sem).wait()
        pltpu.sync_copy(rows_v, out_hbm.at[pl.ds(base, _RPW)])

    return _pe_gather


def kernel(h, w, pe):
    return _build_pe_gather()(pe)[None]


# speedup vs baseline: 1.0177x; 1.0177x over previous
import functools

import jax
import jax.numpy as jnp
from jax import lax
from jax.experimental import pallas as pl
from jax.experimental.pallas import tpu as pltpu, tpu_sc as plsc

N = 1024
D = 768

_NC = 2
_NS = 16
_NW = _NC * _NS
_RPW = N // _NW
_HALF = _RPW // 2


@functools.cache
def _build_pe_copy():
    @functools.partial(
        pl.kernel,
        mesh=plsc.VectorSubcoreMesh(
            core_axis_name="c", subcore_axis_name="s",
            num_cores=_NC, num_subcores=_NS),
        out_type=jax.ShapeDtypeStruct((N, D), jnp.float32),
        scratch_types=[
            pltpu.VMEM((_HALF, D), jnp.float32),
            pltpu.VMEM((_HALF, D), jnp.float32),
            pltpu.SemaphoreType.DMA,
            pltpu.SemaphoreType.DMA,
            pltpu.SemaphoreType.DMA,
            pltpu.SemaphoreType.DMA,
        ],
    )
    def _pe_copy(pe_hbm, out_hbm, buf0, buf1, r0, r1, w0, w1):
        wid = lax.axis_index("s") * _NC + lax.axis_index("c")
        base = wid * _RPW
        rd0 = pltpu.async_copy(pe_hbm.at[pl.ds(base, _HALF)], buf0, r0)
        rd1 = pltpu.async_copy(pe_hbm.at[pl.ds(base + _HALF, _HALF)], buf1, r1)
        rd0.wait()
        wr0 = pltpu.async_copy(buf0, out_hbm.at[pl.ds(base, _HALF)], w0)
        rd1.wait()
        wr1 = pltpu.async_copy(buf1, out_hbm.at[pl.ds(base + _HALF, _HALF)], w1)
        wr0.wait()
        wr1.wait()

    return _pe_copy


def kernel(h, w, pe):
    return _build_pe_copy()(pe)[None]
